# baseline (speedup 1.0000x reference)
import functools

import jax
import jax.numpy as jnp
from jax import lax
from jax.experimental import pallas as pl
from jax.experimental.pallas import tpu as pltpu
from jax.experimental.pallas import tpu_sc as plsc

N = 10000
E = 320000
DIN = 128
B = 64
DSF = 16
DOUT = 10

NC = 2
NS = 16
NW = NC * NS
K = 128
CHUNKS = 80
G = 16
IG = CHUNKS // G
CA = (IG + 1) * G
EPW = CHUNKS * K
EPAD = EPW * NW
RPS = 640
NPAD = RPS * NS
D = 128


def _sc_partial_agg(feat, idxp):
    mesh = plsc.VectorSubcoreMesh(
        core_axis_name="c", subcore_axis_name="s", num_cores=NC, num_subcores=NS
    )

    @functools.partial(
        pl.kernel,
        out_type=jax.ShapeDtypeStruct((NC * NPAD, D), jnp.float32),
        mesh=mesh,
        scratch_types=[
            [pltpu.VMEM((G, 2, K), jnp.int32)] * 2,
            [pltpu.VMEM((K, D), jnp.float32)] * 2,
            pltpu.VMEM((8, D), jnp.float32),
            pltpu.VMEM_SHARED((NPAD, D), jnp.float32),
            [pltpu.SemaphoreType.DMA] * 2,
            [pltpu.SemaphoreType.DMA] * 2,
        ],
    )
    def agg_kernel(feat_hbm, idx_hbm, out_hbm,
                   ibuf, rows, zer_v, acc_sh, isems, gsems):
        cid = lax.axis_index("c")
        sid = lax.axis_index("s")
        wid = sid * NC + cid

        for p in (0, 1):
            pltpu.async_copy(idx_hbm.at[wid, pl.ds(p * G, G)], ibuf[p],
                             isems[p])

        @pl.loop(0, 8)
        def _(r):
            @pl.loop(0, D, step=16)
            def _(c):
                zer_v[r, pl.ds(c, 16)] = jnp.zeros((16,), jnp.float32)

        base = sid * RPS
        pltpu.sync_copy(zer_v, acc_sh.at[pl.ds(base, 8)])
        have = 8
        while have < RPS:
            step = min(have, RPS - have)
            pltpu.sync_copy(acc_sh.at[pl.ds(base, step)],
                            acc_sh.at[pl.ds(base + have, step)])
            have += step
        plsc.subcore_barrier()

        def wait_idx(p):
            pltpu.make_async_copy(idx_hbm.at[0, pl.ds(0, G)], ibuf[p],
                                  isems[p]).wait()

        def wait_gather(b):
            pltpu.make_async_copy(feat_hbm.at[pl.ds(0, K)], rows[b],
                                  gsems[b]).wait()

        def do_group(p, g_next_load):
            wait_idx(p)
            for b in (0, 1):
                pltpu.async_copy(feat_hbm.at[ibuf[p].at[b, 0]], rows[b],
                                 gsems[b])
            for j in range(G):
                b = j % 2
                wait_gather(b)
                pltpu.sync_copy(rows[b], acc_sh.at[ibuf[p].at[j, 1]], add=True)
                if j + 2 < G:
                    pltpu.async_copy(feat_hbm.at[ibuf[p].at[j + 2, 0]],
                                     rows[b], gsems[b])
            if g_next_load is not None:
                pltpu.async_copy(
                    idx_hbm.at[wid, pl.ds(g_next_load * G, G)], ibuf[p],
                    isems[p])

        @pl.loop(0, IG // 2)
        def _(h):
            g = h * 2
            do_group(0, g + 2)
            do_group(1, g + 3)

        if IG % 2:
            do_group(0, None)

        plsc.subcore_barrier()
        pltpu.sync_copy(
            acc_sh.at[pl.ds(sid * RPS, RPS)],
            out_hbm.at[pl.ds(cid * NPAD + sid * RPS, RPS)],
        )

    return agg_kernel(feat, idxp)


def _bn_relu(h, gamma, beta):
    m = jnp.mean(h, axis=0)
    v = jnp.mean((h - m) ** 2, axis=0)
    return jnp.maximum((h - m) * lax.rsqrt(v + 1e-5) * gamma + beta, 0.0)


def _tc_layer1(x, part, w1, b1, bng, bnb, w2, b2, g1, bb1):
    def body(x_ref, p_ref, w1_ref, b1_ref, bng_ref, bnb_ref,
             w2_ref, b2_ref, g1_ref, bb1_ref, out_ref):
        a = x_ref[...] + p_ref[0:N, :] + p_ref[NPAD:NPAD + N, :]
        h = jnp.dot(a, w1_ref[...], preferred_element_type=jnp.float32) + b1_ref[...]
        h = _bn_relu(h, bng_ref[...], bnb_ref[...])
        h = jnp.dot(h, w2_ref[...], preferred_element_type=jnp.float32) + b2_ref[...]
        h = _bn_relu(h, g1_ref[...], bb1_ref[...])
        out_ref[...] = jnp.concatenate(
            [h, jnp.zeros((N, D - h.shape[1]), jnp.float32)], axis=1
        )

    return pl.pallas_call(
        body, out_shape=jax.ShapeDtypeStruct((N, D), jnp.float32)
    )(x, part, w1, b1, bng, bnb, w2, b2, g1, bb1)


def _tc_layer2(h1, part, gids, sf, w1, b1, bng, bnb, w2, b2, g2, bb2,
               f1w, f1b, f2w, f2b):
    def body(h_ref, p_ref, gid_ref, sf_ref, w1_ref, b1_ref, bng_ref, bnb_ref,
             w2_ref, b2_ref, g2_ref, bb2_ref, f1w_ref, f1b_ref, f2w_ref,
             f2b_ref, out_ref):
        a = h_ref[...] + p_ref[0:N, :] + p_ref[NPAD:NPAD + N, :]
        a = a[:, 0:100]
        h = jnp.dot(a, w1_ref[...], preferred_element_type=jnp.float32) + b1_ref[...]
        h = _bn_relu(h, bng_ref[...], bnb_ref[...])
        h = jnp.dot(h, w2_ref[...], preferred_element_type=jnp.float32) + b2_ref[...]
        h = _bn_relu(h, g2_ref[...], bb2_ref[...])
        onehot = (gid_ref[...] == lax.broadcasted_iota(jnp.int32, (1, B), 1))
        onehot = onehot.astype(jnp.float32)
        sums = lax.dot_general(
            onehot, h, (((0,), (0,)), ((), ())),
            preferred_element_type=jnp.float32,
        )
        counts = jnp.sum(onehot, axis=0)
        hg = sums / jnp.maximum(counts, 1.0)[:, None]
        hg = jnp.concatenate([hg, sf_ref[...]], axis=1)
        o = jnp.maximum(
            jnp.dot(hg, f1w_ref[...], preferred_element_type=jnp.float32)
            + f1b_ref[...], 0.0)
        out_ref[...] = (
            jnp.dot(o, f2w_ref[...], preferred_element_type=jnp.float32)
            + f2b_ref[...]
        )

    return pl.pallas_call(
        body, out_shape=jax.ShapeDtypeStruct((B, DOUT), jnp.float32)
    )(h1, part, gids, sf, w1, b1, bng, bnb, w2, b2, g2, bb2, f1w, f1b, f2w, f2b)


def kernel(x, edge_index, graph_ids, self_feat,
           g1_w1, g1_b1, g1_bn_g, g1_bn_b, g1_w2, g1_b2, bn1_g, bn1_b,
           g2_w1, g2_b1, g2_bn_g, g2_bn_b, g2_w2, g2_b2, bn2_g, bn2_b,
           fc1_w, fc1_b, fc2_w, fc2_b):
    pad = EPAD - E
    srcp = jnp.concatenate(
        [edge_index[0], jnp.zeros((pad,), jnp.int32)]).reshape(NW, CHUNKS, K)
    dstp = jnp.concatenate(
        [edge_index[1], jnp.full((pad,), N, jnp.int32)]).reshape(NW, CHUNKS, K)
    idxp = jnp.stack([srcp, dstp], axis=2)
    idxp = jnp.concatenate(
        [idxp, jnp.zeros((NW, CA - CHUNKS, 2, K), jnp.int32)], axis=1)

    part1 = _sc_partial_agg(x, idxp)
    h1 = _tc_layer1(x, part1, g1_w1, g1_b1, g1_bn_g, g1_bn_b,
                    g1_w2, g1_b2, bn1_g, bn1_b)
    part2 = _sc_partial_agg(h1, idxp)
    out = _tc_layer2(h1, part2, graph_ids.reshape(N, 1), self_feat,
                     g2_w1, g2_b1, g2_bn_g, g2_bn_b, g2_w2, g2_b2,
                     bn2_g, bn2_b, fc1_w, fc1_b, fc2_w, fc2_b)
    return out


# speedup vs baseline: 1.5113x; 1.5113x over previous
import functools

import jax
import jax.numpy as jnp
from jax import lax
from jax.experimental import pallas as pl
from jax.experimental.pallas import tpu as pltpu
from jax.experimental.pallas import tpu_sc as plsc

N = 10000
E = 320000
DIN = 128
B = 64
DSF = 16
DOUT = 10

NC = 2
NS = 16
NW = NC * NS
K = 128
CHUNKS = 80
G = 16
IG = CHUNKS // G
CA = (IG + 1) * G
EPW = CHUNKS * K
EPAD = EPW * NW
RPS = 640
NPAD = RPS * NS
D1 = 112
D2 = 32


def _sc_partial_agg(feat, idxp, ds):
    mesh = plsc.VectorSubcoreMesh(
        core_axis_name="c", subcore_axis_name="s", num_cores=NC, num_subcores=NS
    )

    @functools.partial(
        pl.kernel,
        out_type=jax.ShapeDtypeStruct((NC, NPAD, ds), jnp.float32),
        mesh=mesh,
        scratch_types=[
            [pltpu.VMEM((G, 2, K), jnp.int32)] * 2,
            [pltpu.VMEM((K, 128), jnp.float32)] * 2,
            pltpu.VMEM((K, ds), jnp.float32),
            pltpu.VMEM_SHARED((NPAD, ds), jnp.float32),
            [pltpu.SemaphoreType.DMA] * 2,
            [pltpu.SemaphoreType.DMA] * 2,
        ],
    )
    def agg_kernel(feat_hbm, idx_hbm, zero_hbm, out_hbm,
                   ibuf, rows, cmp_v, acc_sh, isems, gsems):
        cid = lax.axis_index("c")
        sid = lax.axis_index("s")
        wid = sid * NC + cid

        for p in (0, 1):
            pltpu.async_copy(idx_hbm.at[wid, pl.ds(p * G, G)], ibuf[p],
                             isems[p])

        @pl.when(sid == 0)
        def _():
            pltpu.sync_copy(zero_hbm, acc_sh)
        plsc.subcore_barrier()

        def wait_idx(p):
            pltpu.make_async_copy(idx_hbm.at[0, pl.ds(0, G)], ibuf[p],
                                  isems[p]).wait()

        def wait_gather(b):
            pltpu.make_async_copy(feat_hbm.at[pl.ds(0, K)], rows[b],
                                  gsems[b]).wait()

        def compact_scatter(p, j, b):
            if ds < 128:
                @pl.loop(0, K)
                def _(r):
                    for c in range(0, ds, 16):
                        cmp_v[r, pl.ds(c, 16)] = rows[b][r, pl.ds(c, 16)]
                pltpu.sync_copy(cmp_v, acc_sh.at[ibuf[p].at[j, 1]], add=True)
            else:
                pltpu.sync_copy(rows[b], acc_sh.at[ibuf[p].at[j, 1]], add=True)

        def do_group(p, g_next_load):
            wait_idx(p)
            for b in (0, 1):
                pltpu.async_copy(feat_hbm.at[ibuf[p].at[b, 0]], rows[b],
                                 gsems[b])

            @pl.loop(0, G // 2 - 1)
            def _(j2):
                for b in (0, 1):
                    j = j2 * 2 + b
                    wait_gather(b)
                    compact_scatter(p, j, b)
                    pltpu.async_copy(feat_hbm.at[ibuf[p].at[j + 2, 0]],
                                     rows[b], gsems[b])

            for b in (0, 1):
                wait_gather(b)
                compact_scatter(p, G - 2 + b, b)

            if g_next_load is not None:
                pltpu.async_copy(
                    idx_hbm.at[wid, pl.ds(g_next_load * G, G)], ibuf[p],
                    isems[p])

        @pl.loop(0, IG // 2)
        def _(h):
            g = h * 2
            do_group(0, g + 2)
            do_group(1, g + 3)

        if IG % 2:
            do_group(0, None)

        plsc.subcore_barrier()
        @pl.when(sid == 0)
        def _():
            pltpu.sync_copy(acc_sh, out_hbm.at[cid])

    return agg_kernel(feat, idxp, jnp.zeros((NPAD, ds), jnp.float32))


def _bn_relu(h, gamma, beta):
    m = jnp.mean(h, axis=0)
    v = jnp.mean((h - m) ** 2, axis=0)
    return jnp.maximum((h - m) * lax.rsqrt(v + 1e-5) * gamma + beta, 0.0)


def _tc_project1(x, w1):
    def body(x_ref, w1_ref, out_ref):
        z = jnp.dot(x_ref[...], w1_ref[...], preferred_element_type=jnp.float32)
        out_ref[...] = jnp.concatenate(
            [z, jnp.zeros((N, 128 - z.shape[1]), jnp.float32)], axis=1)

    return pl.pallas_call(
        body, out_shape=jax.ShapeDtypeStruct((N, 128), jnp.float32))(x, w1)


def _tc_mid(z1, part, b1, bng, bnb, w2, b2, g1, bb1, w2b):
    def body(z_ref, p_ref, b1_ref, bng_ref, bnb_ref, w2_ref, b2_ref,
             g1_ref, bb1_ref, w2b_ref, out_ref):
        h = (z_ref[:, 0:100] + p_ref[0, 0:N, 0:100]
             + p_ref[1, 0:N, 0:100] + b1_ref[...])
        h = _bn_relu(h, bng_ref[...], bnb_ref[...])
        h = jnp.dot(h, w2_ref[...], preferred_element_type=jnp.float32) + b2_ref[...]
        h = _bn_relu(h, g1_ref[...], bb1_ref[...])
        z2 = jnp.dot(h, w2b_ref[...], preferred_element_type=jnp.float32)
        out_ref[...] = jnp.concatenate(
            [z2, jnp.zeros((N, 128 - z2.shape[1]), jnp.float32)], axis=1)

    return pl.pallas_call(
        body, out_shape=jax.ShapeDtypeStruct((N, 128), jnp.float32)
    )(z1, part, b1, bng, bnb, w2, b2, g1, bb1, w2b)


def _tc_final(z2, part, gids, sf, b1, bng, bnb, w2, b2, g2, bb2,
              f1w, f1b, f2w, f2b):
    def body(z_ref, p_ref, gid_ref, sf_ref, b1_ref, bng_ref, bnb_ref,
             w2_ref, b2_ref, g2_ref, bb2_ref, f1w_ref, f1b_ref, f2w_ref,
             f2b_ref, out_ref):
        h = (z_ref[:, 0:20] + p_ref[0, 0:N, 0:20]
             + p_ref[1, 0:N, 0:20] + b1_ref[...])
        h = _bn_relu(h, bng_ref[...], bnb_ref[...])
        h = jnp.dot(h, w2_ref[...], preferred_element_type=jnp.float32) + b2_ref[...]
        h = _bn_relu(h, g2_ref[...], bb2_ref[...])
        onehot = (gid_ref[...] == lax.broadcasted_iota(jnp.int32, (1, B), 1))
        onehot = onehot.astype(jnp.float32)
        sums = lax.dot_general(
            onehot, h, (((0,), (0,)), ((), ())),
            preferred_element_type=jnp.float32,
        )
        counts = jnp.sum(onehot, axis=0)
        hg = sums / jnp.maximum(counts, 1.0)[:, None]
        hg = jnp.concatenate([hg, sf_ref[...]], axis=1)
        o = jnp.maximum(
            jnp.dot(hg, f1w_ref[...], preferred_element_type=jnp.float32)
            + f1b_ref[...], 0.0)
        out_ref[...] = (
            jnp.dot(o, f2w_ref[...], preferred_element_type=jnp.float32)
            + f2b_ref[...]
        )

    return pl.pallas_call(
        body, out_shape=jax.ShapeDtypeStruct((B, DOUT), jnp.float32)
    )(z2, part, gids, sf, b1, bng, bnb, w2, b2, g2, bb2, f1w, f1b, f2w, f2b)


def kernel(x, edge_index, graph_ids, self_feat,
           g1_w1, g1_b1, g1_bn_g, g1_bn_b, g1_w2, g1_b2, bn1_g, bn1_b,
           g2_w1, g2_b1, g2_bn_g, g2_bn_b, g2_w2, g2_b2, bn2_g, bn2_b,
           fc1_w, fc1_b, fc2_w, fc2_b):
    pad = EPAD - E
    srcp = jnp.concatenate(
        [edge_index[0], jnp.zeros((pad,), jnp.int32)]).reshape(NW, CHUNKS, K)
    dstp = jnp.concatenate(
        [edge_index[1], jnp.full((pad,), N, jnp.int32)]).reshape(NW, CHUNKS, K)
    idxp = jnp.stack([srcp, dstp], axis=2)
    idxp = jnp.concatenate(
        [idxp, jnp.zeros((NW, CA - CHUNKS, 2, K), jnp.int32)], axis=1)

    z1 = _tc_project1(x, g1_w1)
    part1 = _sc_partial_agg(z1, idxp, 128)
    z2 = _tc_mid(z1, part1, g1_b1, g1_bn_g, g1_bn_b, g1_w2, g1_b2,
                 bn1_g, bn1_b, g2_w1)
    part2 = _sc_partial_agg(z2, idxp, 128)
    out = _tc_final(z2, part2, graph_ids.reshape(N, 1), self_feat,
                    g2_b1, g2_bn_g, g2_bn_b, g2_w2, g2_b2, bn2_g, bn2_b,
                    fc1_w, fc1_b, fc2_w, fc2_b)
    return out


# speedup vs baseline: 1.5115x; 1.0001x over previous
import functools

import jax
import jax.numpy as jnp
from jax import lax
from jax.experimental import pallas as pl
from jax.experimental.pallas import tpu as pltpu
from jax.experimental.pallas import tpu_sc as plsc

N = 10000
E = 320000
DIN = 128
B = 64
DSF = 16
DOUT = 10

NC = 2
NS = 16
NW = NC * NS
K = 128
CHUNKS = 80
G = 16
IG = CHUNKS // G
CA = (IG + 1) * G
EPW = CHUNKS * K
EPAD = EPW * NW
RPS = 640
NPAD = RPS * NS
DS = 128


def _sc_partial_agg(feat, idxp, ds=DS):
    mesh = plsc.VectorSubcoreMesh(
        core_axis_name="c", subcore_axis_name="s", num_cores=NC, num_subcores=NS
    )

    @functools.partial(
        pl.kernel,
        out_type=jax.ShapeDtypeStruct((NC, NPAD, ds), jnp.float32),
        mesh=mesh,
        scratch_types=[
            [pltpu.VMEM((G, 2, K), jnp.int32)] * 2,
            [pltpu.VMEM((K, 128), jnp.float32)] * 2,
            pltpu.VMEM_SHARED((NPAD, ds), jnp.float32),
            [pltpu.SemaphoreType.DMA] * 2,
            [pltpu.SemaphoreType.DMA] * 2,
        ],
    )
    def agg_kernel(feat_hbm, idx_hbm, zero_hbm, out_hbm,
                   ibuf, rows, acc_sh, isems, gsems):
        cid = lax.axis_index("c")
        sid = lax.axis_index("s")
        wid = sid * NC + cid

        for p in (0, 1):
            pltpu.async_copy(idx_hbm.at[wid, pl.ds(p * G, G)], ibuf[p],
                             isems[p])

        @pl.when(sid == 0)
        def _():
            pltpu.sync_copy(zero_hbm, acc_sh)
        plsc.subcore_barrier()

        def wait_idx(p):
            pltpu.make_async_copy(idx_hbm.at[0, pl.ds(0, G)], ibuf[p],
                                  isems[p]).wait()

        def wait_gather(b):
            pltpu.make_async_copy(feat_hbm.at[pl.ds(0, K)], rows[b],
                                  gsems[b]).wait()

        def do_group(p, g_next_load):
            wait_idx(p)
            for b in (0, 1):
                pltpu.async_copy(feat_hbm.at[ibuf[p].at[b, 0]], rows[b],
                                 gsems[b])

            @pl.loop(0, G // 2 - 1)
            def _(j2):
                for b in (0, 1):
                    j = j2 * 2 + b
                    wait_gather(b)
                    pltpu.sync_copy(rows[b], acc_sh.at[ibuf[p].at[j, 1]],
                                    add=True)
                    pltpu.async_copy(feat_hbm.at[ibuf[p].at[j + 2, 0]],
                                     rows[b], gsems[b])

            for b in (0, 1):
                wait_gather(b)
                pltpu.sync_copy(rows[b], acc_sh.at[ibuf[p].at[G - 2 + b, 1]],
                                add=True)

            if g_next_load is not None:
                pltpu.async_copy(
                    idx_hbm.at[wid, pl.ds(g_next_load * G, G)], ibuf[p],
                    isems[p])

        @pl.loop(0, IG // 2)
        def _(h):
            g = h * 2
            do_group(0, g + 2)
            do_group(1, g + 3)

        if IG % 2:
            do_group(0, None)

        plsc.subcore_barrier()
        @pl.when(sid == 0)
        def _():
            pltpu.sync_copy(acc_sh, out_hbm.at[cid])

    return agg_kernel(feat, idxp, jnp.zeros((NPAD, ds), jnp.float32))


def _bn_relu(h, gamma, beta):
    m = jnp.mean(h, axis=0)
    v = jnp.mean((h - m) ** 2, axis=0)
    return jnp.maximum((h - m) * lax.rsqrt(v + 1e-5) * gamma + beta, 0.0)


def _tc_project1(x, w1):
    def body(x_ref, w1_ref, out_ref):
        z = jnp.dot(x_ref[...], w1_ref[...], preferred_element_type=jnp.float32)
        out_ref[...] = jnp.concatenate(
            [z, jnp.zeros((N, 128 - z.shape[1]), jnp.float32)], axis=1)

    return pl.pallas_call(
        body, out_shape=jax.ShapeDtypeStruct((N, 128), jnp.float32))(x, w1)


def _tc_mid(z1, part, b1, bng, bnb, w2, b2, g1, bb1, w2b):
    def body(z_ref, p_ref, b1_ref, bng_ref, bnb_ref, w2_ref, b2_ref,
             g1_ref, bb1_ref, w2b_ref, out_ref):
        h = (z_ref[:, 0:100] + p_ref[0, 0:N, 0:100]
             + p_ref[1, 0:N, 0:100] + b1_ref[...])
        h = _bn_relu(h, bng_ref[...], bnb_ref[...])
        h = jnp.dot(h, w2_ref[...], preferred_element_type=jnp.float32) + b2_ref[...]
        h = _bn_relu(h, g1_ref[...], bb1_ref[...])
        z2 = jnp.dot(h, w2b_ref[...], preferred_element_type=jnp.float32)
        out_ref[...] = jnp.concatenate(
            [z2, jnp.zeros((N, 128 - z2.shape[1]), jnp.float32)], axis=1)

    return pl.pallas_call(
        body, out_shape=jax.ShapeDtypeStruct((N, 128), jnp.float32)
    )(z1, part, b1, bng, bnb, w2, b2, g1, bb1, w2b)


def _tc_final(z2, part, gids, sf, b1, bng, bnb, w2, b2, g2, bb2,
              f1w, f1b, f2w, f2b):
    def body(z_ref, p_ref, gid_ref, sf_ref, b1_ref, bng_ref, bnb_ref,
             w2_ref, b2_ref, g2_ref, bb2_ref, f1w_ref, f1b_ref, f2w_ref,
             f2b_ref, out_ref):
        h = (z_ref[:, 0:20] + p_ref[0, 0:N, 0:20]
             + p_ref[1, 0:N, 0:20] + b1_ref[...])
        h = _bn_relu(h, bng_ref[...], bnb_ref[...])
        h = jnp.dot(h, w2_ref[...], preferred_element_type=jnp.float32) + b2_ref[...]
        h = _bn_relu(h, g2_ref[...], bb2_ref[...])
        onehot = (gid_ref[...] == lax.broadcasted_iota(jnp.int32, (1, B), 1))
        onehot = onehot.astype(jnp.float32)
        sums = lax.dot_general(
            onehot, h, (((0,), (0,)), ((), ())),
            preferred_element_type=jnp.float32,
        )
        counts = jnp.sum(onehot, axis=0)
        hg = sums / jnp.maximum(counts, 1.0)[:, None]
        hg = jnp.concatenate([hg, sf_ref[...]], axis=1)
        o = jnp.maximum(
            jnp.dot(hg, f1w_ref[...], preferred_element_type=jnp.float32)
            + f1b_ref[...], 0.0)
        out_ref[...] = (
            jnp.dot(o, f2w_ref[...], preferred_element_type=jnp.float32)
            + f2b_ref[...]
        )

    return pl.pallas_call(
        body, out_shape=jax.ShapeDtypeStruct((B, DOUT), jnp.float32)
    )(z2, part, gids, sf, b1, bng, bnb, w2, b2, g2, bb2, f1w, f1b, f2w, f2b)


def kernel(x, edge_index, graph_ids, self_feat,
           g1_w1, g1_b1, g1_bn_g, g1_bn_b, g1_w2, g1_b2, bn1_g, bn1_b,
           g2_w1, g2_b1, g2_bn_g, g2_bn_b, g2_w2, g2_b2, bn2_g, bn2_b,
           fc1_w, fc1_b, fc2_w, fc2_b):
    pad = EPAD - E
    srcp = jnp.concatenate(
        [edge_index[0], jnp.zeros((pad,), jnp.int32)]).reshape(NW, CHUNKS, K)
    dstp = jnp.concatenate(
        [edge_index[1], jnp.full((pad,), N, jnp.int32)]).reshape(NW, CHUNKS, K)
    idxp = jnp.stack([srcp, dstp], axis=2)
    idxp = jnp.concatenate(
        [idxp, jnp.zeros((NW, CA - CHUNKS, 2, K), jnp.int32)], axis=1)

    z1 = _tc_project1(x, g1_w1)
    part1 = _sc_partial_agg(z1, idxp)
    z2 = _tc_mid(z1, part1, g1_b1, g1_bn_g, g1_bn_b, g1_w2, g1_b2,
                 bn1_g, bn1_b, g2_w1)
    part2 = _sc_partial_agg(z2, idxp)
    out = _tc_final(z2, part2, graph_ids.reshape(N, 1), self_feat,
                    g2_b1, g2_bn_g, g2_bn_b, g2_w2, g2_b2, bn2_g, bn2_b,
                    fc1_w, fc1_b, fc2_w, fc2_b)
    return out
